# baseline (speedup 1.0000x reference)
import numpy as np

import jax
import jax.numpy as jnp
from jax import lax
from jax.experimental import pallas as pl
from jax.experimental.pallas import tpu as pltpu
from jax.experimental.pallas import tpu_sc as plsc

B = 4096
S = 200
D = 128
V = 100000
L = 16
NC = 2
NS = 16
NW = NC * NS
BPW = B // NW
DW = D // 2
NDW = DW // L
S0 = 128
S1 = S - S0
_NFULL = S // L
_TAIL = S - L
_TAIL_NEW = _NFULL * L - _TAIL


def _body(ids_hbm, table_hbm, out_hbm, idx_v, rows_v, out_v, sem0, sem1):
    wid = lax.axis_index("s") * NC + lax.axis_index("c")
    base = wid * BPW
    pltpu.sync_copy(ids_hbm.at[pl.ds(base, BPW), :], idx_v)
    sems = (sem0, sem1)

    def start_gather(b, k):
        pltpu.async_copy(table_hbm.at[idx_v.at[b, pl.ds(0, S0)]],
                         rows_v.at[k, pl.ds(0, S0)], sems[k])
        pltpu.async_copy(table_hbm.at[idx_v.at[b, pl.ds(S0, S1)]],
                         rows_v.at[k, pl.ds(S0, S1)], sems[k])

    def wait_gather(b, k):
        pltpu.make_async_copy(table_hbm.at[idx_v.at[b, pl.ds(0, S0)]],
                              rows_v.at[k, pl.ds(0, S0)], sems[k]).wait()
        pltpu.make_async_copy(table_hbm.at[idx_v.at[b, pl.ds(S0, S1)]],
                              rows_v.at[k, pl.ds(S0, S1)], sems[k]).wait()

    vsplat = jnp.full((L,), V, jnp.int32)
    iota = lax.iota(jnp.int32, L)

    def mask_ids(b):
        cnt = jnp.zeros((L,), jnp.int32)
        chunks = []
        for c in range(_NFULL):
            v = idx_v[b, pl.ds(c * L, L)]
            chunks.append(v)
            cnt = cnt + plsc.all_reduce_population_count(v != 0)
        vt = idx_v[b, pl.ds(_TAIL, L)]
        fresh = (vt != 0) & (iota >= _TAIL_NEW)
        cnt = cnt + plsc.all_reduce_population_count(fresh)
        for c in range(_NFULL):
            pos = iota + (c * L)
            idx_v[b, pl.ds(c * L, L)] = jnp.where(pos < cnt, chunks[c], vsplat)
        pos = iota + _TAIL
        idx_v[b, pl.ds(_TAIL, L)] = jnp.where(pos < cnt, vt, vsplat)
        return cnt

    hi_mask = jnp.full((L,), np.int32(-65536), jnp.int32)

    def compute(b, k, len_vec):
        acc0 = (jnp.zeros((L,), jnp.float32),) * (2 * NDW)

        def add_body(s, acc):
            new = []
            for w in range(NDW):
                x = rows_v[k, s, pl.ds(w * L, L)]
                ev = plsc.bitcast(x << 16, jnp.float32)
                od = plsc.bitcast(x & hi_mask, jnp.float32)
                new.append(acc[2 * w] + ev)
                new.append(acc[2 * w + 1] + od)
            return tuple(new)

        acc = lax.fori_loop(0, S, add_body, acc0)
        inv_v = 1.0 / len_vec.astype(jnp.float32)
        for w in range(NDW):
            out_v[b, pl.ds(w * 2 * L, L)] = acc[2 * w] * inv_v
            out_v[b, pl.ds(w * 2 * L + L, L)] = acc[2 * w + 1] * inv_v

    len_cur = mask_ids(0)
    start_gather(0, 0)

    def outer(i, len_cur):
        for j in range(2):
            b = i * 2 + j
            bn = jnp.minimum(b + 1, BPW - 1)
            len_next = mask_ids(bn)

            @pl.when(b + 1 < BPW)
            def _():
                start_gather(b + 1, (j + 1) % 2)

            wait_gather(b, j)
            compute(b, j, len_cur)
            len_cur = len_next
        return len_cur

    lax.fori_loop(0, BPW // 2, outer, len_cur)
    pltpu.sync_copy(out_v, out_hbm.at[pl.ds(base, BPW), :])


def _col_perm():
    j = np.arange(L)
    blk = np.empty(2 * L, np.int64)
    blk[2 * j] = j
    blk[2 * j + 1] = L + j
    return np.concatenate([2 * L * k + blk for k in range(D // (2 * L))])


_PERM = _col_perm()


def kernel(input_ids, table):
    ids = input_ids.astype(jnp.int32)
    tb = table.astype(jnp.bfloat16)[:, _PERM]
    tb = jnp.concatenate([tb, jnp.zeros((1, D), jnp.bfloat16)], axis=0)
    tb_i32 = lax.bitcast_convert_type(tb.reshape(V + 1, DW, 2), jnp.int32)
    mesh = plsc.VectorSubcoreMesh(core_axis_name="c", subcore_axis_name="s")
    f = pl.kernel(
        _body,
        out_type=jax.ShapeDtypeStruct((B, D), jnp.float32),
        mesh=mesh,
        compiler_params=pltpu.CompilerParams(needs_layout_passes=False,
                                             use_tc_tiling_on_sc=False),
        scratch_types=[
            pltpu.VMEM((BPW, S), jnp.int32),
            pltpu.VMEM((2, S, DW), jnp.int32),
            pltpu.VMEM((BPW, D), jnp.float32),
            pltpu.SemaphoreType.DMA,
            pltpu.SemaphoreType.DMA,
        ],
    )
    return f(ids, tb_i32)


# speedup vs baseline: 2.0496x; 2.0496x over previous
import numpy as np

import jax
import jax.numpy as jnp
from jax import lax
from jax.experimental import pallas as pl
from jax.experimental.pallas import tpu as pltpu
from jax.experimental.pallas import tpu_sc as plsc

B = 4096
S = 200
D = 128
V = 100000
L = 16
NC = 2
NS = 16
NW = NC * NS
BPW = B // NW
DW = D // 2
NDW = DW // L
S0 = 128
S1 = S - S0
_NFULL = S // L
_TAIL = S - L
_TAIL_NEW = _NFULL * L - _TAIL


def _body(ids_hbm, table_hbm, out_hbm, idx_v, rows_v, out_v, sem0, sem1):
    wid = lax.axis_index("s") * NC + lax.axis_index("c")
    base = wid * BPW
    pltpu.sync_copy(ids_hbm.at[pl.ds(base * S, BPW * S)], idx_v)
    sems = (sem0, sem1)

    def start_gather(b, k):
        pltpu.async_copy(table_hbm.at[idx_v.at[pl.ds(b * S, S0)]],
                         rows_v.at[k, pl.ds(0, S0)], sems[k])
        pltpu.async_copy(table_hbm.at[idx_v.at[pl.ds(b * S + S0, S1)]],
                         rows_v.at[k, pl.ds(S0, S1)], sems[k])

    def wait_gather(b, k):
        pltpu.make_async_copy(table_hbm.at[idx_v.at[pl.ds(b * S, S0)]],
                              rows_v.at[k, pl.ds(0, S0)], sems[k]).wait()
        pltpu.make_async_copy(table_hbm.at[idx_v.at[pl.ds(b * S + S0, S1)]],
                              rows_v.at[k, pl.ds(S0, S1)], sems[k]).wait()

    vsplat = jnp.full((L,), V, jnp.int32)
    iota = lax.iota(jnp.int32, L)

    def mask_ids(b):
        cnt = jnp.zeros((L,), jnp.int32)
        chunks = []
        for c in range(_NFULL):
            v = idx_v[pl.ds(b * S + c * L, L)]
            chunks.append(v)
            cnt = cnt + plsc.all_reduce_population_count(v != 0)
        vt = idx_v[pl.ds(b * S + _TAIL, L)]
        fresh = (vt != 0) & (iota >= _TAIL_NEW)
        cnt = cnt + plsc.all_reduce_population_count(fresh)
        for c in range(_NFULL):
            pos = iota + (c * L)
            idx_v[pl.ds(b * S + c * L, L)] = jnp.where(pos < cnt, chunks[c],
                                                       vsplat)
        pos = iota + _TAIL
        idx_v[pl.ds(b * S + _TAIL, L)] = jnp.where(pos < cnt, vt, vsplat)
        return cnt

    hi_mask = jnp.full((L,), np.int32(-65536), jnp.int32)

    def compute(b, k, len_vec):
        acc0 = (jnp.zeros((L,), jnp.float32),) * (2 * NDW)

        def add_body(s, acc):
            new = []
            for w in range(NDW):
                x = rows_v[k, s, pl.ds(w * L, L)]
                ev = plsc.bitcast(x << 16, jnp.float32)
                od = plsc.bitcast(x & hi_mask, jnp.float32)
                new.append(acc[2 * w] + ev)
                new.append(acc[2 * w + 1] + od)
            return tuple(new)

        acc = lax.fori_loop(0, S, add_body, acc0)
        inv_v = 1.0 / len_vec.astype(jnp.float32)
        for w in range(NDW):
            out_v[pl.ds(b * D + w * 2 * L, L)] = acc[2 * w] * inv_v
            out_v[pl.ds(b * D + w * 2 * L + L, L)] = acc[2 * w + 1] * inv_v

    len_cur = mask_ids(0)
    start_gather(0, 0)

    def outer(i, len_cur):
        for j in range(2):
            b = i * 2 + j
            bn = jnp.minimum(b + 1, BPW - 1)
            len_next = mask_ids(bn)

            @pl.when(b + 1 < BPW)
            def _():
                start_gather(b + 1, (j + 1) % 2)

            wait_gather(b, j)
            compute(b, j, len_cur)
            len_cur = len_next
        return len_cur

    lax.fori_loop(0, BPW // 2, outer, len_cur)
    pltpu.sync_copy(out_v, out_hbm.at[pl.ds(base * D, BPW * D)])


def _pack_table(table):
    x = lax.bitcast_convert_type(table, jnp.uint32).reshape(V, NDW, 2, L)
    ev = x[:, :, 0, :]
    od = x[:, :, 1, :]
    rne = jnp.uint32(0x7FFF)
    one = jnp.uint32(1)
    ev_b = (ev + rne + ((ev >> 16) & one)) >> 16
    od_b = (od + rne + ((od >> 16) & one)) & jnp.uint32(0xFFFF0000)
    packed = (od_b | ev_b).reshape(V, DW)
    packed = jnp.concatenate(
        [packed, jnp.zeros((1, DW), jnp.uint32)], axis=0)
    return lax.bitcast_convert_type(packed, jnp.int32)


def kernel(input_ids, table):
    ids = input_ids.astype(jnp.int32).reshape(B * S)
    tb_i32 = _pack_table(table.astype(jnp.float32))
    mesh = plsc.VectorSubcoreMesh(core_axis_name="c", subcore_axis_name="s")
    f = pl.kernel(
        _body,
        out_type=jax.ShapeDtypeStruct((B * D,), jnp.float32),
        mesh=mesh,
        compiler_params=pltpu.CompilerParams(needs_layout_passes=False,
                                             use_tc_tiling_on_sc=False),
        scratch_types=[
            pltpu.VMEM((BPW * S,), jnp.int32),
            pltpu.VMEM((2, S, DW), jnp.int32),
            pltpu.VMEM((BPW * D,), jnp.float32),
            pltpu.SemaphoreType.DMA,
            pltpu.SemaphoreType.DMA,
        ],
    )
    return f(ids, tb_i32).reshape(B, D)


# speedup vs baseline: 3.3369x; 1.6281x over previous
import jax
import jax.numpy as jnp
from jax import lax
from jax.experimental import pallas as pl
from jax.experimental.pallas import tpu as pltpu
from jax.experimental.pallas import tpu_sc as plsc

B = 4096
S = 200
D = 128
L = 16
NC = 2
NS = 16
NW = NC * NS
BPW = B // NW
ND = D // L
S0 = 128
S1 = S - S0
_NFULL = S // L
_TAIL = S - L
_TAIL_NEW = _NFULL * L - _TAIL


def _encoder_body(ids_hbm, table_hbm, out_hbm, idx_v, rows_v, out_v, sem0, sem1):
    wid = lax.axis_index("s") * NC + lax.axis_index("c")
    base = wid * BPW
    pltpu.sync_copy(ids_hbm.at[pl.ds(base, BPW), :], idx_v)

    sems = (sem0, sem1)

    def start_gather(b, k):
        pltpu.async_copy(table_hbm.at[idx_v.at[b, pl.ds(0, S0)]],
                         rows_v.at[k, pl.ds(0, S0)], sems[k])
        pltpu.async_copy(table_hbm.at[idx_v.at[b, pl.ds(S0, S1)]],
                         rows_v.at[k, pl.ds(S0, S1)], sems[k])

    def wait_gather(b, k):
        pltpu.make_async_copy(table_hbm.at[idx_v.at[b, pl.ds(0, S0)]],
                              rows_v.at[k, pl.ds(0, S0)], sems[k]).wait()
        pltpu.make_async_copy(table_hbm.at[idx_v.at[b, pl.ds(S0, S1)]],
                              rows_v.at[k, pl.ds(S0, S1)], sems[k]).wait()

    def seq_len_splat(b):
        cnt = jnp.zeros((L,), jnp.int32)
        for c in range(_NFULL):
            v = idx_v[b, pl.ds(c * L, L)]
            cnt = cnt + plsc.all_reduce_population_count(v != 0)
        v = idx_v[b, pl.ds(_TAIL, L)]
        fresh = (v != 0) & (lax.iota(jnp.int32, L) >= _TAIL_NEW)
        cnt = cnt + plsc.all_reduce_population_count(fresh)
        return cnt

    def compute(b, k):
        len_vec = seq_len_splat(b)
        fzero = jnp.zeros((L,), jnp.float32)
        acc0 = (fzero,) * ND

        def add_body(s, acc):
            m = jnp.full((L,), s, jnp.int32) < len_vec
            return tuple(acc[d] + jnp.where(m, rows_v[k, s, pl.ds(d * L, L)],
                                            fzero)
                         for d in range(ND))

        acc = lax.fori_loop(0, S, add_body, acc0)
        inv_v = 1.0 / len_vec.astype(jnp.float32)
        for d in range(ND):
            out_v[b, pl.ds(d * L, L)] = acc[d] * inv_v

    start_gather(0, 0)

    def outer(i, carry):
        for j in range(2):
            b = i * 2 + j

            @pl.when(b + 1 < BPW)
            def _():
                start_gather(b + 1, (j + 1) % 2)

            wait_gather(b, j)
            compute(b, j)
        return carry

    lax.fori_loop(0, BPW // 2, outer, 0)
    pltpu.sync_copy(out_v, out_hbm.at[pl.ds(base, BPW), :])


def kernel(input_ids, table):
    ids = input_ids.astype(jnp.int32)
    table = table.astype(jnp.float32)
    mesh = plsc.VectorSubcoreMesh(core_axis_name="c", subcore_axis_name="s")
    f = pl.kernel(
        _encoder_body,
        out_type=jax.ShapeDtypeStruct((B, D), jnp.float32),
        mesh=mesh,
        compiler_params=pltpu.CompilerParams(needs_layout_passes=False),
        scratch_types=[
            pltpu.VMEM((BPW, S), jnp.int32),
            pltpu.VMEM((2, S, D), jnp.float32),
            pltpu.VMEM((BPW, D), jnp.float32),
            pltpu.SemaphoreType.DMA,
            pltpu.SemaphoreType.DMA,
        ],
    )
    return f(ids, table)


# speedup vs baseline: 3.3469x; 1.0030x over previous
import jax
import jax.numpy as jnp
from jax import lax
from jax.experimental import pallas as pl
from jax.experimental.pallas import tpu as pltpu
from jax.experimental.pallas import tpu_sc as plsc

B = 4096
S = 200
D = 128
L = 16
NC = 2
NS = 16
NW = NC * NS
BPW = B // NW
ND = D // L
S0 = 128
S1 = S - S0
_NFULL = S // L
_TAIL = S - L
_TAIL_NEW = _NFULL * L - _TAIL


def _encoder_body(ids_hbm, table_hbm, out_hbm, idx_v, rows_v, out_v, sem0, sem1):
    wid = lax.axis_index("s") * NC + lax.axis_index("c")
    base = wid * BPW
    pltpu.sync_copy(ids_hbm.at[pl.ds(base, BPW), :], idx_v)

    sems = (sem0, sem1)

    _SPLITS = ((0, 64), (64, 64), (128, 64), (192, 8))

    def start_gather(b, k):
        for off, n in _SPLITS:
            pltpu.async_copy(table_hbm.at[idx_v.at[b, pl.ds(off, n)]],
                             rows_v.at[k, pl.ds(off, n)], sems[k])

    def wait_gather(b, k):
        for off, n in _SPLITS:
            pltpu.make_async_copy(table_hbm.at[idx_v.at[b, pl.ds(off, n)]],
                                  rows_v.at[k, pl.ds(off, n)], sems[k]).wait()

    def seq_len_splat(b):
        cnt = jnp.zeros((L,), jnp.int32)
        for c in range(_NFULL):
            v = idx_v[b, pl.ds(c * L, L)]
            cnt = cnt + plsc.all_reduce_population_count(v != 0)
        v = idx_v[b, pl.ds(_TAIL, L)]
        fresh = (v != 0) & (lax.iota(jnp.int32, L) >= _TAIL_NEW)
        cnt = cnt + plsc.all_reduce_population_count(fresh)
        return cnt

    def compute(b, k):
        len_vec = seq_len_splat(b)
        fzero = jnp.zeros((L,), jnp.float32)
        acc0 = (fzero,) * ND

        def add_body(s, acc):
            m = jnp.full((L,), s, jnp.int32) < len_vec
            return tuple(acc[d] + jnp.where(m, rows_v[k, s, pl.ds(d * L, L)],
                                            fzero)
                         for d in range(ND))

        acc = lax.fori_loop(0, S, add_body, acc0)
        inv_v = 1.0 / len_vec.astype(jnp.float32)
        for d in range(ND):
            out_v[b, pl.ds(d * L, L)] = acc[d] * inv_v

    start_gather(0, 0)

    def outer(i, carry):
        for j in range(2):
            b = i * 2 + j

            @pl.when(b + 1 < BPW)
            def _():
                start_gather(b + 1, (j + 1) % 2)

            wait_gather(b, j)
            compute(b, j)
        return carry

    lax.fori_loop(0, BPW // 2, outer, 0)
    pltpu.sync_copy(out_v, out_hbm.at[pl.ds(base, BPW), :])


def kernel(input_ids, table):
    ids = input_ids.astype(jnp.int32)
    table = table.astype(jnp.float32)
    mesh = plsc.VectorSubcoreMesh(core_axis_name="c", subcore_axis_name="s")
    f = pl.kernel(
        _encoder_body,
        out_type=jax.ShapeDtypeStruct((B, D), jnp.float32),
        mesh=mesh,
        compiler_params=pltpu.CompilerParams(needs_layout_passes=False),
        scratch_types=[
            pltpu.VMEM((BPW, S), jnp.int32),
            pltpu.VMEM((2, S, D), jnp.float32),
            pltpu.VMEM((BPW, D), jnp.float32),
            pltpu.SemaphoreType.DMA,
            pltpu.SemaphoreType.DMA,
        ],
    )
    return f(ids, table)
